# Initial kernel scaffold; baseline (speedup 1.0000x reference)
import jax
import jax.numpy as jnp
from jax.experimental import pallas as pl


def kernel(inp, arc_source, arc_target, W1, b1, W2, b2, W3, b3, W4, b4):
    raise NotImplementedError("write your pallas kernel here")


# speedup vs baseline: 7.2808x; 7.2808x over previous
import jax
import jax.numpy as jnp
from jax import lax
from jax.experimental import pallas as pl
from jax.experimental.pallas import tpu as pltpu
from jax.experimental.pallas import tpu_sc as plsc

N_NODES = 100000
E_EDGES = 1600000
MAX_IT = 5

_NC, _NS, _LANES = 2, 16, 16
_NW = _NC * _NS
_EPAD = 1638400
_PT = _EPAD // _NW
_CH = 2048
_NCHUNK = _PT // _CH
_SUB = 128
_NSUB = _CH // _SUB
_NPAD = 106496
_RT = _NPAD // _NS
_BE = 4096
_BN = 4096
_NBLK = 25


def _base_body(x_ref, w_ref, b_ref, o_ref):
    o_ref[...] = jnp.dot(x_ref[...], w_ref[...],
                         preferred_element_type=jnp.float32) + b_ref[0:1, 0:8]


def _compute_basep(inp_pad, w1x, b1p8):
    return pl.pallas_call(
        _base_body,
        grid=(_EPAD // _BE,),
        in_specs=[
            pl.BlockSpec((_BE, 16), lambda i: (i, 0)),
            pl.BlockSpec((16, 8), lambda i: (0, 0)),
            pl.BlockSpec((8, 128), lambda i: (0, 0)),
        ],
        out_specs=pl.BlockSpec((_BE, 8), lambda i: (i, 0)),
        out_shape=jax.ShapeDtypeStruct((_EPAD, 8), jnp.float32),
    )(inp_pad, w1x, b1p8)


def _sc_iter_body(b0h, b1h, b2h, b3h, b4h, src_hbm, tgt_hbm, prev_hbm, w_hbm,
                  out_hbm,
                  w_v, src_v, tgt_v, bv0, bv1, bv2, bv3, bv4,
                  gv0, gv1, gv2, gv3, mv0, mv1, mv2, mv3, a_v, b_v,
                  c0, c1, c2, c3, n0, n1, n2, n3, gsem, ssem):
    c = lax.axis_index("c")
    s = lax.axis_index("s")
    wid = c * _NS + s
    bhs = [b0h, b1h, b2h, b3h, b4h]
    bvs = [bv0, bv1, bv2, bv3, bv4]
    gvs = [gv0, gv1, gv2, gv3]
    mvs = [mv0, mv1, mv2, mv3]
    curs = [c0, c1, c2, c3]
    news = [n0, n1, n2, n3]

    pltpu.sync_copy(w_hbm, w_v)
    zero16 = jnp.zeros((_LANES,), jnp.float32)

    def _stageA():
        row0 = s * _RT
        for comp in range(4):
            pltpu.sync_copy(prev_hbm.at[pl.ds(comp * _NPAD + row0, _RT)], a_v)
            pltpu.sync_copy(
                prev_hbm.at[pl.ds((4 + comp) * _NPAD + row0, _RT)], b_v)

            def _add(i, _):
                a_v[pl.ds(i * _LANES, _LANES)] = (
                    a_v[pl.ds(i * _LANES, _LANES)]
                    + b_v[pl.ds(i * _LANES, _LANES)])
                return _

            lax.fori_loop(0, _RT // _LANES, _add, None)
            pltpu.sync_copy(a_v, curs[comp].at[pl.ds(row0, _RT)])

        def _zero(i, _):
            a_v[pl.ds(i * _LANES, _LANES)] = zero16
            return _

        lax.fori_loop(0, _RT // _LANES, _zero, None)
        for comp in range(4):
            pltpu.sync_copy(a_v, news[comp].at[pl.ds(row0, _RT)])

    wvecs = [w_v[pl.ds(16 * t, 16)] for t in range(4)]

    def _ws(i):
        return wvecs[i // 16][i % 16]

    w1p = [[_ws(c0_ * 5 + j) for j in range(5)] for c0_ in range(4)]
    w2p = [[_ws(20 + j * 4 + k) for k in range(4)] for j in range(5)]
    b2p = [_ws(40 + k) for k in range(4)]

    def _grp(g, _):
        gg = [gvs[cc][pl.ds(g * _LANES, _LANES)] for cc in range(4)]
        bb = [bvs[jj][pl.ds(g * _LANES, _LANES)] for jj in range(5)]
        hs = []
        for j in range(5):
            a = bb[j]
            for c0_ in range(4):
                a = a + gg[c0_] * w1p[c0_][j]
            a = jnp.clip(a, -80.0, 80.0)
            e = jnp.exp(a)
            hs.append((1.0 - e) / (1.0 + e))
        for k in range(4):
            a2 = hs[0] * w2p[0][k]
            for j in range(1, 5):
                a2 = a2 + hs[j] * w2p[j][k]
            a2 = a2 + b2p[k]
            a2 = jnp.clip(a2, -80.0, 80.0)
            e2 = jnp.exp(a2)
            mvs[k][pl.ds(g * _LANES, _LANES)] = (1.0 - e2) / (1.0 + e2)
        return _

    def _stageB():
        e0 = wid * _PT
        r0 = wid * (_PT // _SUB)

        def _chunk(i, _):
            crow = r0 + i * _NSUB
            pltpu.sync_copy(src_hbm.at[pl.ds(crow, _NSUB)], src_v)
            pltpu.sync_copy(tgt_hbm.at[pl.ds(crow, _NSUB)], tgt_v)
            for jj in range(5):
                pltpu.sync_copy(bhs[jj].at[pl.ds(e0 + i * _CH, _CH)], bvs[jj])
            descs = []
            for j in range(_NSUB):
                for cc in range(4):
                    descs.append(pltpu.async_copy(
                        curs[cc].at[src_v.at[j]],
                        gvs[cc].at[pl.ds(j * _SUB, _SUB)], gsem))
            for d in descs:
                d.wait()
            lax.fori_loop(0, _CH // _LANES, _grp, None)
            descs = []
            for j in range(_NSUB):
                for cc in range(4):
                    descs.append(pltpu.async_copy(
                        mvs[cc].at[pl.ds(j * _SUB, _SUB)],
                        news[cc].at[tgt_v.at[j]], ssem, add=True))
            for d in descs:
                d.wait()
            return _

        lax.fori_loop(0, _NCHUNK, _chunk, None)

    def _stageC():
        row0 = s * _RT
        for comp in range(4):
            pltpu.sync_copy(news[comp].at[pl.ds(row0, _RT)], a_v)
            pltpu.sync_copy(
                a_v,
                out_hbm.at[pl.ds((c * 4 + comp) * _NPAD + row0, _RT)])

    _stageA()
    plsc.subcore_barrier()
    _stageB()
    plsc.subcore_barrier()
    _stageC()


def _make_sc_iter():
    mesh = plsc.VectorSubcoreMesh(core_axis_name="c", subcore_axis_name="s",
                                  num_cores=_NC, num_subcores=_NS)
    return pl.kernel(
        _sc_iter_body,
        out_type=jax.ShapeDtypeStruct((8 * _NPAD,), jnp.float32),
        mesh=mesh,
        scratch_types=(
            [pltpu.VMEM((64,), jnp.float32)]
            + [pltpu.VMEM((_NSUB, _SUB), jnp.int32)] * 2
            + [pltpu.VMEM((_CH,), jnp.float32)] * 5
            + [pltpu.VMEM((_CH,), jnp.float32)] * 4
            + [pltpu.VMEM((_CH,), jnp.float32)] * 4
            + [pltpu.VMEM((_RT,), jnp.float32)] * 2
            + [pltpu.VMEM_SHARED((_NPAD,), jnp.float32)] * 8
            + [pltpu.SemaphoreType.DMA] * 2
        ),
        compiler_params=pltpu.CompilerParams(needs_layout_passes=False,
                                             use_tc_tiling_on_sc=False),
    )


def _out_body(p00, p01, p02, p03, p10, p11, p12, p13, w_ref, o_ref):
    ss = [p00[...] + p10[...], p01[...] + p11[...],
          p02[...] + p12[...], p03[...] + p13[...]]
    o1 = []
    for j in range(5):
        a = w_ref[20 + j]
        acc = ss[0] * w_ref[0 * 5 + j]
        for cc in range(1, 4):
            acc = acc + ss[cc] * w_ref[cc * 5 + j]
        o1.append(jnp.tanh(acc + a))
    zs = []
    for k in range(7):
        acc = o1[0] * w_ref[25 + 0 * 7 + k]
        for j in range(1, 5):
            acc = acc + o1[j] * w_ref[25 + j * 7 + k]
        zs.append(acc + w_ref[60 + k])
    m = zs[0]
    for k in range(1, 7):
        m = jnp.maximum(m, zs[k])
    es = [jnp.exp(z - m) for z in zs]
    tot = es[0]
    for k in range(1, 7):
        tot = tot + es[k]
    for k in range(7):
        o_ref[k, :] = es[k] / tot


def _compute_out(p, wout):
    specs = [pl.BlockSpec((_BN,), lambda i, cc=c4: (cc * (_NPAD // _BN) + i,))
             for c4 in range(8)]
    specs.append(pl.BlockSpec(memory_space=pltpu.SMEM))
    return pl.pallas_call(
        _out_body,
        grid=(_NBLK,),
        in_specs=specs,
        out_specs=pl.BlockSpec((7, _BN), lambda i: (0, i)),
        out_shape=jax.ShapeDtypeStruct((7, _NBLK * _BN), jnp.float32),
    )(*([p] * 8), wout)


def kernel(inp, arc_source, arc_target, W1, b1, W2, b2, W3, b3, W4, b4):
    f32 = jnp.float32
    w1x = jnp.zeros((16, 8), f32).at[1:16, :5].set(-2.0 * W1[:15])
    b1p8 = jnp.zeros((8, 128), f32).at[0, 0:5].set(-2.0 * b1)
    inp_pad = jnp.concatenate(
        [inp, jnp.zeros((_EPAD - E_EDGES, 16), f32)])
    basep = _compute_basep(inp_pad, w1x, b1p8)
    bcols = [basep[:, j] for j in range(5)]

    npad = _EPAD - E_EDGES
    pad_idx = jnp.full((npad,), N_NODES, jnp.int32)
    srcs = jnp.concatenate([arc_source.astype(jnp.int32), pad_idx])
    tgts = jnp.concatenate([arc_target.astype(jnp.int32), pad_idx])
    srcs = srcs.reshape(_EPAD // _SUB, _SUB)
    tgts = tgts.reshape(_EPAD // _SUB, _SUB)

    w_all = jnp.concatenate([
        (-2.0 * W1[15:]).ravel(), (-2.0 * W2).ravel(), -2.0 * b2,
        jnp.zeros((20,), f32),
    ])

    wout = jnp.concatenate([
        W3.ravel(), b3, W4.ravel(), b4, jnp.zeros((13,), f32),
    ])

    sc_iter = _make_sc_iter()
    p = jnp.zeros((8 * _NPAD,), f32)
    for _ in range(MAX_IT):
        p = sc_iter(*bcols, srcs, tgts, p, w_all)
    outp = _compute_out(p, wout)
    return outp[:, :N_NODES].T


# speedup vs baseline: 12.4967x; 1.7164x over previous
import jax
import jax.numpy as jnp
from jax import lax
from jax.experimental import pallas as pl
from jax.experimental.pallas import tpu as pltpu
from jax.experimental.pallas import tpu_sc as plsc

N_NODES = 100000
E_EDGES = 1600000
MAX_IT = 5

_NC, _NS, _LANES = 2, 16, 16
_NW = _NC * _NS
_EPAD = 1638400
_PT = _EPAD // _NW
_CH = 2048
_NCHUNK = _PT // _CH
_SUB = 128
_NSUB = _CH // _SUB
_NPAD = 106496
_RT = _NPAD // _NS
_BE = 16384
_BN = 4096
_NBLK = 25


def _base_body(x_ref, w_ref, b_ref, o_ref):
    res = lax.dot_general(w_ref[...], x_ref[...],
                          (((0,), (1,)), ((), ())),
                          preferred_element_type=jnp.float32)
    o_ref[...] = res + b_ref[0:8, 0:1]


def _compute_basep(inp, w1x, b1p8):
    nblk = (E_EDGES + _BE - 1) // _BE
    return pl.pallas_call(
        _base_body,
        grid=(nblk,),
        in_specs=[
            pl.BlockSpec((_BE, 16), lambda i: (i, 0)),
            pl.BlockSpec((16, 8), lambda i: (0, 0)),
            pl.BlockSpec((8, 128), lambda i: (0, 0)),
        ],
        out_specs=pl.BlockSpec((8, _BE), lambda i: (0, i)),
        out_shape=jax.ShapeDtypeStruct((8, _EPAD), jnp.float32),
    )(inp, w1x, b1p8)


def _sc_iter_body(base_hbm, src_hbm, tgt_hbm, prev_hbm, w_hbm,
                  out_hbm,
                  w_v, src_v, tgt_v, bv0, bv1, bv2, bv3, bv4,
                  gv0, gv1, gv2, gv3, mv0, mv1, mv2, mv3, a_v, b_v,
                  c0, c1, c2, c3, n0, n1, n2, n3, gsem, ssem):
    c = lax.axis_index("c")
    s = lax.axis_index("s")
    wid = c * _NS + s
    bvs = [bv0, bv1, bv2, bv3, bv4]
    gvs = [gv0, gv1, gv2, gv3]
    mvs = [mv0, mv1, mv2, mv3]
    curs = [c0, c1, c2, c3]
    news = [n0, n1, n2, n3]

    pltpu.sync_copy(w_hbm, w_v)
    zero16 = jnp.zeros((_LANES,), jnp.float32)

    def _stageA():
        row0 = s * _RT
        for comp in range(4):
            pltpu.sync_copy(prev_hbm.at[pl.ds(comp * _NPAD + row0, _RT)], a_v)
            pltpu.sync_copy(
                prev_hbm.at[pl.ds((4 + comp) * _NPAD + row0, _RT)], b_v)

            def _add(i, _):
                a_v[pl.ds(i * _LANES, _LANES)] = (
                    a_v[pl.ds(i * _LANES, _LANES)]
                    + b_v[pl.ds(i * _LANES, _LANES)])
                return _

            lax.fori_loop(0, _RT // _LANES, _add, None)
            pltpu.sync_copy(a_v, curs[comp].at[pl.ds(row0, _RT)])

        def _zero(i, _):
            a_v[pl.ds(i * _LANES, _LANES)] = zero16
            return _

        lax.fori_loop(0, _RT // _LANES, _zero, None)
        for comp in range(4):
            pltpu.sync_copy(a_v, news[comp].at[pl.ds(row0, _RT)])

    wvecs = [w_v[pl.ds(16 * t, 16)] for t in range(4)]

    def _ws(i):
        return wvecs[i // 16][i % 16]

    w1p = [[_ws(c0_ * 5 + j) for j in range(5)] for c0_ in range(4)]
    w2p = [[_ws(20 + j * 4 + k) for k in range(4)] for j in range(5)]
    b2p = [_ws(40 + k) for k in range(4)]

    def _grp(g, _):
        gg = [gvs[cc][pl.ds(g * _LANES, _LANES)] for cc in range(4)]
        bb = [bvs[jj][pl.ds(g * _LANES, _LANES)] for jj in range(5)]
        hs = []
        for j in range(5):
            a = bb[j]
            for c0_ in range(4):
                a = a + gg[c0_] * w1p[c0_][j]
            a = jnp.clip(a, -80.0, 80.0)
            e = jnp.exp(a)
            hs.append((1.0 - e) / (1.0 + e))
        for k in range(4):
            a2 = hs[0] * w2p[0][k]
            for j in range(1, 5):
                a2 = a2 + hs[j] * w2p[j][k]
            a2 = a2 + b2p[k]
            a2 = jnp.clip(a2, -80.0, 80.0)
            e2 = jnp.exp(a2)
            mvs[k][pl.ds(g * _LANES, _LANES)] = (1.0 - e2) / (1.0 + e2)
        return _

    def _stageB():
        e0 = wid * _PT
        r0 = wid * (_PT // _SUB)

        def _chunk(i, _):
            crow = r0 + i * _NSUB
            pltpu.sync_copy(src_hbm.at[pl.ds(crow, _NSUB)], src_v)
            pltpu.sync_copy(tgt_hbm.at[pl.ds(crow, _NSUB)], tgt_v)
            for jj in range(5):
                pltpu.sync_copy(
                    base_hbm.at[pl.ds(jj * _EPAD + e0 + i * _CH, _CH)],
                    bvs[jj])
            descs = []
            for j in range(_NSUB):
                for cc in range(4):
                    descs.append(pltpu.async_copy(
                        curs[cc].at[src_v.at[j]],
                        gvs[cc].at[pl.ds(j * _SUB, _SUB)], gsem))
            for d in descs:
                d.wait()
            lax.fori_loop(0, _CH // _LANES, _grp, None)
            descs = []
            for j in range(_NSUB):
                for cc in range(4):
                    descs.append(pltpu.async_copy(
                        mvs[cc].at[pl.ds(j * _SUB, _SUB)],
                        news[cc].at[tgt_v.at[j]], ssem, add=True))
            for d in descs:
                d.wait()
            return _

        lax.fori_loop(0, _NCHUNK, _chunk, None)

    def _stageC():
        row0 = s * _RT
        for comp in range(4):
            pltpu.sync_copy(news[comp].at[pl.ds(row0, _RT)], a_v)
            pltpu.sync_copy(
                a_v,
                out_hbm.at[pl.ds((c * 4 + comp) * _NPAD + row0, _RT)])

    _stageA()
    plsc.subcore_barrier()
    _stageB()
    plsc.subcore_barrier()
    _stageC()


def _make_sc_iter():
    mesh = plsc.VectorSubcoreMesh(core_axis_name="c", subcore_axis_name="s",
                                  num_cores=_NC, num_subcores=_NS)
    return pl.kernel(
        _sc_iter_body,
        out_type=jax.ShapeDtypeStruct((8 * _NPAD,), jnp.float32),
        mesh=mesh,
        scratch_types=(
            [pltpu.VMEM((64,), jnp.float32)]
            + [pltpu.VMEM((_NSUB, _SUB), jnp.int32)] * 2
            + [pltpu.VMEM((_CH,), jnp.float32)] * 5
            + [pltpu.VMEM((_CH,), jnp.float32)] * 4
            + [pltpu.VMEM((_CH,), jnp.float32)] * 4
            + [pltpu.VMEM((_RT,), jnp.float32)] * 2
            + [pltpu.VMEM_SHARED((_NPAD,), jnp.float32)] * 8
            + [pltpu.SemaphoreType.DMA] * 2
        ),
        compiler_params=pltpu.CompilerParams(needs_layout_passes=False,
                                             use_tc_tiling_on_sc=False),
    )


def _out_body(p00, p01, p02, p03, p10, p11, p12, p13, w_ref, o_ref):
    ss = [p00[...] + p10[...], p01[...] + p11[...],
          p02[...] + p12[...], p03[...] + p13[...]]
    o1 = []
    for j in range(5):
        a = w_ref[20 + j]
        acc = ss[0] * w_ref[0 * 5 + j]
        for cc in range(1, 4):
            acc = acc + ss[cc] * w_ref[cc * 5 + j]
        o1.append(jnp.tanh(acc + a))
    zs = []
    for k in range(7):
        acc = o1[0] * w_ref[25 + 0 * 7 + k]
        for j in range(1, 5):
            acc = acc + o1[j] * w_ref[25 + j * 7 + k]
        zs.append(acc + w_ref[60 + k])
    m = zs[0]
    for k in range(1, 7):
        m = jnp.maximum(m, zs[k])
    es = [jnp.exp(z - m) for z in zs]
    tot = es[0]
    for k in range(1, 7):
        tot = tot + es[k]
    for k in range(7):
        o_ref[k, :] = es[k] / tot


def _compute_out(p, wout):
    specs = [pl.BlockSpec((_BN,), lambda i, cc=c4: (cc * (_NPAD // _BN) + i,))
             for c4 in range(8)]
    specs.append(pl.BlockSpec(memory_space=pltpu.SMEM))
    return pl.pallas_call(
        _out_body,
        grid=(_NBLK,),
        in_specs=specs,
        out_specs=pl.BlockSpec((7, _BN), lambda i: (0, i)),
        out_shape=jax.ShapeDtypeStruct((7, _NBLK * _BN), jnp.float32),
    )(*([p] * 8), wout)


def kernel(inp, arc_source, arc_target, W1, b1, W2, b2, W3, b3, W4, b4):
    f32 = jnp.float32
    w1x = jnp.zeros((16, 8), f32).at[1:16, :5].set(-2.0 * W1[:15])
    b1p8 = jnp.zeros((8, 128), f32).at[0:5, 0].set(-2.0 * b1)
    basep = _compute_basep(inp, w1x, b1p8)
    baseflat = basep.reshape(8 * _EPAD)

    npad = _EPAD - E_EDGES
    pad_idx = jnp.full((npad,), N_NODES, jnp.int32)
    srcs = jnp.concatenate([arc_source.astype(jnp.int32), pad_idx])
    tgts = jnp.concatenate([arc_target.astype(jnp.int32), pad_idx])
    srcs = srcs.reshape(_EPAD // _SUB, _SUB)
    tgts = tgts.reshape(_EPAD // _SUB, _SUB)

    w_all = jnp.concatenate([
        (-2.0 * W1[15:]).ravel(), (-2.0 * W2).ravel(), -2.0 * b2,
        jnp.zeros((20,), f32),
    ])

    wout = jnp.concatenate([
        W3.ravel(), b3, W4.ravel(), b4, jnp.zeros((13,), f32),
    ])

    sc_iter = _make_sc_iter()
    p = jnp.zeros((8 * _NPAD,), f32)
    for _ in range(MAX_IT):
        p = sc_iter(baseflat, srcs, tgts, p, w_all)
    outp = _compute_out(p, wout)
    return outp[:, :N_NODES].T


# speedup vs baseline: 14.4732x; 1.1582x over previous
import jax
import jax.numpy as jnp
from jax import lax
from jax.experimental import pallas as pl
from jax.experimental.pallas import tpu as pltpu
from jax.experimental.pallas import tpu_sc as plsc

N_NODES = 100000
E_EDGES = 1600000
MAX_IT = 5

_NC, _NS, _LANES = 2, 16, 16
_NW = _NC * _NS
_EPAD = 1638400
_PT = _EPAD // _NW
_CH = 2048
_NCHUNK = _PT // _CH
_SUB = 128
_NSUB = _CH // _SUB
_NPAD = 106496
_RT = _NPAD // _NS
_BE = 16384
_BN = 4096
_NBLK = 25


def _base_body(x_ref, w_ref, b_ref, o0, o1, o2, o3, o4):
    res = lax.dot_general(w_ref[...], x_ref[...],
                          (((0,), (1,)), ((), ())),
                          preferred_element_type=jnp.float32)
    res = res + b_ref[0:8, 0:1]
    for j, o in enumerate((o0, o1, o2, o3, o4)):
        o[...] = res[j]


def _compute_basep(inp, w1x, b1p8):
    nblk = (E_EDGES + _BE - 1) // _BE
    return pl.pallas_call(
        _base_body,
        grid=(nblk,),
        in_specs=[
            pl.BlockSpec((_BE, 16), lambda i: (i, 0)),
            pl.BlockSpec((16, 8), lambda i: (0, 0)),
            pl.BlockSpec((8, 128), lambda i: (0, 0)),
        ],
        out_specs=[pl.BlockSpec((_BE,), lambda i: (i,))] * 5,
        out_shape=[jax.ShapeDtypeStruct((_EPAD,), jnp.float32)] * 5,
    )(inp, w1x, b1p8)


def _sc_iter_body(b0h, b1h, b2h, b3h, b4h, src_hbm, tgt_hbm, prev_hbm, w_hbm,
                  out_hbm,
                  w_v,
                  srcA, tgtA, bA0, bA1, bA2, bA3, bA4,
                  gA0, gA1, gA2, gA3, mA0, mA1, mA2, mA3,
                  srcB, tgtB, bB0, bB1, bB2, bB3, bB4,
                  gB0, gB1, gB2, gB3, mB0, mB1, mB2, mB3,
                  a_v, b_v,
                  c0, c1, c2, c3, n0, n1, n2, n3,
                  lsemA, gsemA, ssemA, lsemB, gsemB, ssemB):
    c = lax.axis_index("c")
    s = lax.axis_index("s")
    wid = c * _NS + s
    bhs = [b0h, b1h, b2h, b3h, b4h]
    setA = (srcA, tgtA, [bA0, bA1, bA2, bA3, bA4],
            [gA0, gA1, gA2, gA3], [mA0, mA1, mA2, mA3],
            lsemA, gsemA, ssemA)
    setB = (srcB, tgtB, [bB0, bB1, bB2, bB3, bB4],
            [gB0, gB1, gB2, gB3], [mB0, mB1, mB2, mB3],
            lsemB, gsemB, ssemB)
    curs = [c0, c1, c2, c3]
    news = [n0, n1, n2, n3]

    pltpu.sync_copy(w_hbm, w_v)
    zero16 = jnp.zeros((_LANES,), jnp.float32)

    def _stageA():
        row0 = s * _RT
        for comp in range(4):
            pltpu.sync_copy(prev_hbm.at[pl.ds(comp * _NPAD + row0, _RT)], a_v)
            pltpu.sync_copy(
                prev_hbm.at[pl.ds((4 + comp) * _NPAD + row0, _RT)], b_v)

            def _add(i, _):
                a_v[pl.ds(i * _LANES, _LANES)] = (
                    a_v[pl.ds(i * _LANES, _LANES)]
                    + b_v[pl.ds(i * _LANES, _LANES)])
                return _

            lax.fori_loop(0, _RT // _LANES, _add, None)
            pltpu.sync_copy(a_v, curs[comp].at[pl.ds(row0, _RT)])

        def _zero(i, _):
            a_v[pl.ds(i * _LANES, _LANES)] = zero16
            return _

        lax.fori_loop(0, _RT // _LANES, _zero, None)
        for comp in range(4):
            pltpu.sync_copy(a_v, news[comp].at[pl.ds(row0, _RT)])

    wvecs = [w_v[pl.ds(16 * t, 16)] for t in range(4)]

    def _ws(i):
        return wvecs[i // 16][i % 16]

    w1p = [[_ws(c0_ * 5 + j) for j in range(5)] for c0_ in range(4)]
    w2p = [[_ws(20 + j * 4 + k) for k in range(4)] for j in range(5)]
    b2p = [_ws(40 + k) for k in range(4)]

    def _mk_grp(bvsL, gvsL, mvsL):
        def _grp(g, _):
            gg = [gvsL[cc][pl.ds(g * _LANES, _LANES)] for cc in range(4)]
            bb = [bvsL[jj][pl.ds(g * _LANES, _LANES)] for jj in range(5)]
            hs = []
            for j in range(5):
                a = bb[j]
                for c0_ in range(4):
                    a = a + gg[c0_] * w1p[c0_][j]
                a = jnp.clip(a, -80.0, 80.0)
                e = jnp.exp(a)
                hs.append((1.0 - e) / (1.0 + e))
            for k in range(4):
                a2 = hs[0] * w2p[0][k]
                for j in range(1, 5):
                    a2 = a2 + hs[j] * w2p[j][k]
                a2 = a2 + b2p[k]
                a2 = jnp.clip(a2, -80.0, 80.0)
                e2 = jnp.exp(a2)
                mvsL[k][pl.ds(g * _LANES, _LANES)] = (1.0 - e2) / (1.0 + e2)
            return _
        return _grp

    def _stageB():
        e0 = wid * _PT
        r0 = wid * (_PT // _SUB)

        def _front(i, st):
            srcv, tgtv, bvsL, gvsL, mvsL, lsem, gsem, _ = st
            crow = r0 + i * _NSUB
            dl = [pltpu.async_copy(src_hbm.at[pl.ds(crow, _NSUB)], srcv, lsem),
                  pltpu.async_copy(tgt_hbm.at[pl.ds(crow, _NSUB)], tgtv, lsem)]
            db = [pltpu.async_copy(bhs[jj].at[pl.ds(e0 + i * _CH, _CH)],
                                   bvsL[jj], lsem) for jj in range(5)]
            dl[0].wait()
            dl[1].wait()
            gds = []
            for j in range(_NSUB):
                for cc in range(4):
                    gds.append(pltpu.async_copy(
                        curs[cc].at[srcv.at[j]],
                        gvsL[cc].at[pl.ds(j * _SUB, _SUB)], gsem))
            for d in db:
                d.wait()
            for d in gds:
                d.wait()
            lax.fori_loop(0, _CH // _LANES, _mk_grp(bvsL, gvsL, mvsL), None)

        def _fire_scatters(st):
            srcv, tgtv, bvsL, gvsL, mvsL, _, _, ssem = st
            sds = []
            for j in range(_NSUB):
                for cc in range(4):
                    sds.append(pltpu.async_copy(
                        mvsL[cc].at[pl.ds(j * _SUB, _SUB)],
                        news[cc].at[tgtv.at[j]], ssem, add=True))
            return sds

        def _pair(t, _):
            iA = 2 * t
            _front(iA, setA)
            sdsA = _fire_scatters(setA)
            _front(iA + 1, setB)
            sdsB = _fire_scatters(setB)
            for d in sdsA:
                d.wait()
            for d in sdsB:
                d.wait()
            return _

        lax.fori_loop(0, _NCHUNK // 2, _pair, None)
        _front(_NCHUNK - 1, setA)
        for d in _fire_scatters(setA):
            d.wait()

    def _stageC():
        row0 = s * _RT
        for comp in range(4):
            pltpu.sync_copy(news[comp].at[pl.ds(row0, _RT)], a_v)
            pltpu.sync_copy(
                a_v,
                out_hbm.at[pl.ds((c * 4 + comp) * _NPAD + row0, _RT)])

    _stageA()
    plsc.subcore_barrier()
    _stageB()
    plsc.subcore_barrier()
    _stageC()


def _make_sc_iter():
    mesh = plsc.VectorSubcoreMesh(core_axis_name="c", subcore_axis_name="s",
                                  num_cores=_NC, num_subcores=_NS)
    return pl.kernel(
        _sc_iter_body,
        out_type=jax.ShapeDtypeStruct((8 * _NPAD,), jnp.float32),
        mesh=mesh,
        scratch_types=(
            [pltpu.VMEM((64,), jnp.float32)]
            + ([pltpu.VMEM((_NSUB, _SUB), jnp.int32)] * 2
               + [pltpu.VMEM((_CH,), jnp.float32)] * 13) * 2
            + [pltpu.VMEM((_RT,), jnp.float32)] * 2
            + [pltpu.VMEM_SHARED((_NPAD,), jnp.float32)] * 8
            + [pltpu.SemaphoreType.DMA] * 6
        ),
        compiler_params=pltpu.CompilerParams(needs_layout_passes=False,
                                             use_tc_tiling_on_sc=False),
    )


def _out_body(p00, p01, p02, p03, p10, p11, p12, p13, w_ref, o_ref):
    ss = [p00[...] + p10[...], p01[...] + p11[...],
          p02[...] + p12[...], p03[...] + p13[...]]
    o1 = []
    for j in range(5):
        a = w_ref[20 + j]
        acc = ss[0] * w_ref[0 * 5 + j]
        for cc in range(1, 4):
            acc = acc + ss[cc] * w_ref[cc * 5 + j]
        o1.append(jnp.tanh(acc + a))
    zs = []
    for k in range(7):
        acc = o1[0] * w_ref[25 + 0 * 7 + k]
        for j in range(1, 5):
            acc = acc + o1[j] * w_ref[25 + j * 7 + k]
        zs.append(acc + w_ref[60 + k])
    m = zs[0]
    for k in range(1, 7):
        m = jnp.maximum(m, zs[k])
    es = [jnp.exp(z - m) for z in zs]
    tot = es[0]
    for k in range(1, 7):
        tot = tot + es[k]
    for k in range(7):
        o_ref[k, :] = es[k] / tot


def _compute_out(p, wout):
    specs = [pl.BlockSpec((_BN,), lambda i, cc=c4: (cc * (_NPAD // _BN) + i,))
             for c4 in range(8)]
    specs.append(pl.BlockSpec(memory_space=pltpu.SMEM))
    return pl.pallas_call(
        _out_body,
        grid=(_NBLK,),
        in_specs=specs,
        out_specs=pl.BlockSpec((7, _BN), lambda i: (0, i)),
        out_shape=jax.ShapeDtypeStruct((7, _NBLK * _BN), jnp.float32),
    )(*([p] * 8), wout)


def kernel(inp, arc_source, arc_target, W1, b1, W2, b2, W3, b3, W4, b4):
    f32 = jnp.float32
    w1x = jnp.zeros((16, 8), f32).at[1:16, :5].set(-2.0 * W1[:15])
    b1p8 = jnp.zeros((8, 128), f32).at[0:5, 0].set(-2.0 * b1)
    bcols = _compute_basep(inp, w1x, b1p8)

    npad = _EPAD - E_EDGES
    pad_idx = jnp.full((npad,), N_NODES, jnp.int32)
    srcs = jnp.concatenate([arc_source.astype(jnp.int32), pad_idx])
    tgts = jnp.concatenate([arc_target.astype(jnp.int32), pad_idx])
    srcs = srcs.reshape(_EPAD // _SUB, _SUB)
    tgts = tgts.reshape(_EPAD // _SUB, _SUB)

    w_all = jnp.concatenate([
        (-2.0 * W1[15:]).ravel(), (-2.0 * W2).ravel(), -2.0 * b2,
        jnp.zeros((20,), f32),
    ])

    wout = jnp.concatenate([
        W3.ravel(), b3, W4.ravel(), b4, jnp.zeros((13,), f32),
    ])

    sc_iter = _make_sc_iter()
    p = jnp.zeros((8 * _NPAD,), f32)
    for _ in range(MAX_IT):
        p = sc_iter(*bcols, srcs, tgts, p, w_all)
    outp = _compute_out(p, wout)
    return outp[:, :N_NODES].T


# speedup vs baseline: 17.6979x; 1.2228x over previous
import jax
import jax.numpy as jnp
from jax import lax
from jax.experimental import pallas as pl
from jax.experimental.pallas import tpu as pltpu
from jax.experimental.pallas import tpu_sc as plsc

N_NODES = 100000
E_EDGES = 1600000
MAX_IT = 5

_NC, _NS, _LANES = 2, 16, 16
_NW = _NC * _NS
_EPAD = 1638400
_PT = _EPAD // _NW
_CH = 2048
_NCHUNK = _PT // _CH
_SUB = 128
_NSUB = _CH // _SUB
_NPAD = 106496
_RT = _NPAD // _NS
_BE = 16384
_BN = 4096
_NBLK = 25


def _base_body(x_ref, w_ref, b_ref, o0, o1, o2, o3, o4):
    res = lax.dot_general(w_ref[...], x_ref[...],
                          (((0,), (0,)), ((), ())),
                          preferred_element_type=jnp.float32)
    res = res + b_ref[0:8, 0:1]
    for j, o in enumerate((o0, o1, o2, o3, o4)):
        o[...] = res[j]


def _compute_basep(inp, w1x, b1p8):
    nblk = (E_EDGES + _BE - 1) // _BE
    return pl.pallas_call(
        _base_body,
        grid=(nblk,),
        in_specs=[
            pl.BlockSpec((16, _BE), lambda i: (0, i)),
            pl.BlockSpec((16, 8), lambda i: (0, 0)),
            pl.BlockSpec((8, 128), lambda i: (0, 0)),
        ],
        out_specs=[pl.BlockSpec((_BE,), lambda i: (i,))] * 5,
        out_shape=[jax.ShapeDtypeStruct((_EPAD,), jnp.float32)] * 5,
    )(inp, w1x, b1p8)


def _sc_iter_body(b0h, b1h, b2h, b3h, b4h, src_hbm, tgt_hbm, prev_hbm, w_hbm,
                  out_hbm,
                  w_v,
                  srcA, tgtA, bA0, bA1, bA2, bA3, bA4,
                  gA0, gA1, gA2, gA3, mA0, mA1, mA2, mA3,
                  srcB, tgtB, bB0, bB1, bB2, bB3, bB4,
                  gB0, gB1, gB2, gB3, mB0, mB1, mB2, mB3,
                  a_v, b_v,
                  c0, c1, c2, c3, n0, n1, n2, n3,
                  lsemA, gsemA, ssemA, lsemB, gsemB, ssemB):
    c = lax.axis_index("c")
    s = lax.axis_index("s")
    wid = c * _NS + s
    bhs = [b0h, b1h, b2h, b3h, b4h]
    setA = (srcA, tgtA, [bA0, bA1, bA2, bA3, bA4],
            [gA0, gA1, gA2, gA3], [mA0, mA1, mA2, mA3],
            lsemA, gsemA, ssemA)
    setB = (srcB, tgtB, [bB0, bB1, bB2, bB3, bB4],
            [gB0, gB1, gB2, gB3], [mB0, mB1, mB2, mB3],
            lsemB, gsemB, ssemB)
    curs = [c0, c1, c2, c3]
    news = [n0, n1, n2, n3]

    pltpu.sync_copy(w_hbm, w_v)
    zero16 = jnp.zeros((_LANES,), jnp.float32)

    def _stageA():
        row0 = s * _RT
        for comp in range(4):
            pltpu.sync_copy(prev_hbm.at[pl.ds(comp * _NPAD + row0, _RT)], a_v)
            pltpu.sync_copy(
                prev_hbm.at[pl.ds((4 + comp) * _NPAD + row0, _RT)], b_v)

            def _add(i, _):
                a_v[pl.ds(i * _LANES, _LANES)] = (
                    a_v[pl.ds(i * _LANES, _LANES)]
                    + b_v[pl.ds(i * _LANES, _LANES)])
                return _

            lax.fori_loop(0, _RT // _LANES, _add, None)
            pltpu.sync_copy(a_v, curs[comp].at[pl.ds(row0, _RT)])

        def _zero(i, _):
            a_v[pl.ds(i * _LANES, _LANES)] = zero16
            return _

        lax.fori_loop(0, _RT // _LANES, _zero, None)
        for comp in range(4):
            pltpu.sync_copy(a_v, news[comp].at[pl.ds(row0, _RT)])

    wvecs = [w_v[pl.ds(16 * t, 16)] for t in range(4)]

    def _ws(i):
        return wvecs[i // 16][i % 16]

    w1p = [[_ws(c0_ * 5 + j) for j in range(5)] for c0_ in range(4)]
    w2p = [[_ws(20 + j * 4 + k) for k in range(4)] for j in range(5)]
    b2p = [_ws(40 + k) for k in range(4)]

    def _mk_grp(bvsL, gvsL, mvsL):
        def _grp(g, _):
            gg = [gvsL[cc][pl.ds(g * _LANES, _LANES)] for cc in range(4)]
            bb = [bvsL[jj][pl.ds(g * _LANES, _LANES)] for jj in range(5)]
            hs = []
            for j in range(5):
                a = bb[j]
                for c0_ in range(4):
                    a = a + gg[c0_] * w1p[c0_][j]
                a = jnp.clip(a, -80.0, 80.0)
                e = jnp.exp(a)
                hs.append((1.0 - e) / (1.0 + e))
            for k in range(4):
                a2 = hs[0] * w2p[0][k]
                for j in range(1, 5):
                    a2 = a2 + hs[j] * w2p[j][k]
                a2 = a2 + b2p[k]
                a2 = jnp.clip(a2, -80.0, 80.0)
                e2 = jnp.exp(a2)
                mvsL[k][pl.ds(g * _LANES, _LANES)] = (1.0 - e2) / (1.0 + e2)
            return _
        return _grp

    def _stageB():
        e0 = wid * _PT
        r0 = wid * (_PT // _SUB)

        def _front(i, st):
            srcv, tgtv, bvsL, gvsL, mvsL, lsem, gsem, _ = st
            crow = r0 + i * _NSUB
            dl = [pltpu.async_copy(src_hbm.at[pl.ds(crow, _NSUB)], srcv, lsem),
                  pltpu.async_copy(tgt_hbm.at[pl.ds(crow, _NSUB)], tgtv, lsem)]
            db = [pltpu.async_copy(bhs[jj].at[pl.ds(e0 + i * _CH, _CH)],
                                   bvsL[jj], lsem) for jj in range(5)]
            dl[0].wait()
            dl[1].wait()
            gds = []
            for j in range(_NSUB):
                for cc in range(4):
                    gds.append(pltpu.async_copy(
                        curs[cc].at[srcv.at[j]],
                        gvsL[cc].at[pl.ds(j * _SUB, _SUB)], gsem))
            for d in db:
                d.wait()
            for d in gds:
                d.wait()
            lax.fori_loop(0, _CH // _LANES, _mk_grp(bvsL, gvsL, mvsL), None)

        def _fire_scatters(st):
            srcv, tgtv, bvsL, gvsL, mvsL, _, _, ssem = st
            sds = []
            for j in range(_NSUB):
                for cc in range(4):
                    sds.append(pltpu.async_copy(
                        mvsL[cc].at[pl.ds(j * _SUB, _SUB)],
                        news[cc].at[tgtv.at[j]], ssem, add=True))
            return sds

        def _pair(t, _):
            iA = 2 * t
            _front(iA, setA)
            sdsA = _fire_scatters(setA)
            _front(iA + 1, setB)
            sdsB = _fire_scatters(setB)
            for d in sdsA:
                d.wait()
            for d in sdsB:
                d.wait()
            return _

        lax.fori_loop(0, _NCHUNK // 2, _pair, None)
        _front(_NCHUNK - 1, setA)
        for d in _fire_scatters(setA):
            d.wait()

    def _stageC():
        row0 = s * _RT
        for comp in range(4):
            pltpu.sync_copy(news[comp].at[pl.ds(row0, _RT)], a_v)
            pltpu.sync_copy(
                a_v,
                out_hbm.at[pl.ds((c * 4 + comp) * _NPAD + row0, _RT)])

    _stageA()
    plsc.subcore_barrier()
    _stageB()
    plsc.subcore_barrier()
    _stageC()


def _make_sc_iter():
    mesh = plsc.VectorSubcoreMesh(core_axis_name="c", subcore_axis_name="s",
                                  num_cores=_NC, num_subcores=_NS)
    return pl.kernel(
        _sc_iter_body,
        out_type=jax.ShapeDtypeStruct((8 * _NPAD,), jnp.float32),
        mesh=mesh,
        scratch_types=(
            [pltpu.VMEM((64,), jnp.float32)]
            + ([pltpu.VMEM((_NSUB, _SUB), jnp.int32)] * 2
               + [pltpu.VMEM((_CH,), jnp.float32)] * 13) * 2
            + [pltpu.VMEM((_RT,), jnp.float32)] * 2
            + [pltpu.VMEM_SHARED((_NPAD,), jnp.float32)] * 8
            + [pltpu.SemaphoreType.DMA] * 6
        ),
        compiler_params=pltpu.CompilerParams(needs_layout_passes=False,
                                             use_tc_tiling_on_sc=False),
    )


def _out_body(p00, p01, p02, p03, p10, p11, p12, p13, w_ref, o_ref):
    ss = [p00[...] + p10[...], p01[...] + p11[...],
          p02[...] + p12[...], p03[...] + p13[...]]
    o1 = []
    for j in range(5):
        a = w_ref[20 + j]
        acc = ss[0] * w_ref[0 * 5 + j]
        for cc in range(1, 4):
            acc = acc + ss[cc] * w_ref[cc * 5 + j]
        o1.append(jnp.tanh(acc + a))
    zs = []
    for k in range(7):
        acc = o1[0] * w_ref[25 + 0 * 7 + k]
        for j in range(1, 5):
            acc = acc + o1[j] * w_ref[25 + j * 7 + k]
        zs.append(acc + w_ref[60 + k])
    m = zs[0]
    for k in range(1, 7):
        m = jnp.maximum(m, zs[k])
    es = [jnp.exp(z - m) for z in zs]
    tot = es[0]
    for k in range(1, 7):
        tot = tot + es[k]
    for k in range(7):
        o_ref[k, :] = es[k] / tot


def _compute_out(p, wout):
    specs = [pl.BlockSpec((_BN,), lambda i, cc=c4: (cc * (_NPAD // _BN) + i,))
             for c4 in range(8)]
    specs.append(pl.BlockSpec(memory_space=pltpu.SMEM))
    return pl.pallas_call(
        _out_body,
        grid=(_NBLK,),
        in_specs=specs,
        out_specs=pl.BlockSpec((7, _BN), lambda i: (0, i)),
        out_shape=jax.ShapeDtypeStruct((7, _NBLK * _BN), jnp.float32),
    )(*([p] * 8), wout)


def kernel(inp, arc_source, arc_target, W1, b1, W2, b2, W3, b3, W4, b4):
    f32 = jnp.float32
    w1x = jnp.zeros((16, 8), f32).at[1:16, :5].set(-2.0 * W1[:15])
    b1p8 = jnp.zeros((8, 128), f32).at[0:5, 0].set(-2.0 * b1)
    bcols = _compute_basep(inp.T, w1x, b1p8)

    npad = _EPAD - E_EDGES
    pad_idx = jnp.full((npad,), N_NODES, jnp.int32)
    srcs = jnp.concatenate([arc_source.astype(jnp.int32), pad_idx])
    tgts = jnp.concatenate([arc_target.astype(jnp.int32), pad_idx])
    srcs = srcs.reshape(_EPAD // _SUB, _SUB)
    tgts = tgts.reshape(_EPAD // _SUB, _SUB)

    w_all = jnp.concatenate([
        (-2.0 * W1[15:]).ravel(), (-2.0 * W2).ravel(), -2.0 * b2,
        jnp.zeros((20,), f32),
    ])

    wout = jnp.concatenate([
        W3.ravel(), b3, W4.ravel(), b4, jnp.zeros((13,), f32),
    ])

    sc_iter = _make_sc_iter()
    p = jnp.zeros((8 * _NPAD,), f32)
    for _ in range(MAX_IT):
        p = sc_iter(*bcols, srcs, tgts, p, w_all)
    outp = _compute_out(p, wout)
    return outp[:, :N_NODES].T
